# baseline (speedup 1.0000x reference)
import functools

import jax
import jax.numpy as jnp
import numpy as np
from jax.experimental import pallas as pl
from jax.experimental.pallas import tpu as pltpu


_NBITS = 16


def _p2_kernel(adj_ref, feat_ref, wg_ref, b_ref, w2_ref, pack_ref,
               p2_ref, words_ref):
    adj = adj_ref[...]
    t = jnp.dot(adj, feat_ref[...],
                preferred_element_type=jnp.float32)
    h = jnp.dot(t, wg_ref[...],
                preferred_element_type=jnp.float32) + b_ref[...]
    emb = jnp.maximum(h, 0.0)
    p2_ref[...] = jnp.dot(emb, w2_ref[...],
                          preferred_element_type=jnp.float32)
    m01 = jnp.where(adj != 0.0, 1.0, 0.0)
    wf = jnp.dot(m01, pack_ref[...],
                 preferred_element_type=jnp.float32)
    words_ref[...] = wf.astype(jnp.int32).astype(jnp.int16)


def _combine_kernel(words_ref, p2_ref, lam_ref, out_ref, *, w_cols):
    w32 = words_ref[...].astype(jnp.int32)
    p2 = p2_ref[...]
    ep2 = jnp.exp(p2 - jnp.max(p2))

    es = []
    den = None
    deg = None
    for bit in range(_NBITS):
        m_b = ((w32 >> bit) & 1).astype(jnp.float32)
        e_b = m_b * ep2[:, bit * w_cols:(bit + 1) * w_cols]
        es.append(e_b)
        s = jnp.sum(e_b, axis=1, keepdims=True)
        d = jnp.sum(m_b, axis=1, keepdims=True)
        den = s if den is None else den + s
        deg = d if deg is None else deg + d

    rowval = pl.reciprocal(jnp.where(deg > 0.0, deg, 1.0))
    scale = lam_ref[0] * pl.reciprocal(jnp.where(den > 0.0, den, 1.0))
    for bit in range(_NBITS):
        e_b = es[bit]
        out_ref[:, bit * w_cols:(bit + 1) * w_cols] = jnp.where(
            e_b > 0.0, rowval + e_b * scale, 0.0)


def _row_tile(n, cap):
    if n % 8 != 0:
        return n
    tm = min(n, cap)
    tm = max(8, (tm // 8) * 8)
    while tm > 8 and n % tm != 0:
        tm -= 8
    return tm if n % tm == 0 else n


def _pack_matrix(n):
    w_cols = n // _NBITS
    wvals = np.array([float(1 << b) for b in range(_NBITS - 1)] + [-32768.0],
                     np.float32)
    p = np.zeros((n, w_cols), np.float32)
    j = np.arange(n)
    p[j, j % w_cols] = wvals[j // w_cols]
    return jnp.asarray(p)


def kernel(v_ori, feat, v_indices, w_gcn, b_gcn, w_mlp, b_mlp, com_lambda):
    del v_indices, b_mlp
    N, F = feat.shape
    H = w_gcn.shape[1]
    w_cols = N // _NBITS

    tm1 = _row_tile(N, 512)
    tm2 = _row_tile(N, 512)

    cp = pltpu.CompilerParams(dimension_semantics=("parallel",),
                              vmem_limit_bytes=(64 << 20) * 3 // 4)
    vmem_full = pl.BlockSpec(memory_space=pltpu.MemorySpace.VMEM)
    smem_full = pl.BlockSpec(memory_space=pltpu.MemorySpace.SMEM)

    w2 = w_mlp.reshape(2, H)[1].reshape(H, 1)
    b = b_gcn.reshape(1, H)
    pack = _pack_matrix(N)

    p2, words = pl.pallas_call(
        _p2_kernel,
        out_shape=(
            jax.ShapeDtypeStruct((N, 1), jnp.float32),
            jax.ShapeDtypeStruct((N, w_cols), jnp.int16),
        ),
        grid=(N // tm1,),
        in_specs=[
            pl.BlockSpec((tm1, N), lambda i: (i, 0)),
            vmem_full,
            vmem_full,
            vmem_full,
            vmem_full,
            vmem_full,
        ],
        out_specs=(
            pl.BlockSpec((tm1, 1), lambda i: (i, 0)),
            pl.BlockSpec((tm1, w_cols), lambda i: (i, 0)),
        ),
        compiler_params=cp,
        cost_estimate=pl.CostEstimate(
            flops=2 * N * N * F + 2 * N * N * w_cols // 8 + 2 * N * F * H,
            transcendentals=0,
            bytes_accessed=4 * (N * N + N * F + F * H + 2 * N) + 2 * N * w_cols),
    )(v_ori, feat, w_gcn, b, w2, pack)

    p2_row = p2.reshape(1, N)
    lam = jnp.asarray(com_lambda, jnp.float32).reshape(1)

    out = pl.pallas_call(
        functools.partial(_combine_kernel, w_cols=w_cols),
        out_shape=jax.ShapeDtypeStruct((N, N), jnp.float32),
        grid=(N // tm2,),
        in_specs=[
            pl.BlockSpec((tm2, w_cols), lambda i: (i, 0)),
            vmem_full,
            smem_full,
        ],
        out_specs=pl.BlockSpec((tm2, N), lambda i: (i, 0)),
        compiler_params=cp,
        cost_estimate=pl.CostEstimate(
            flops=9 * N * N, transcendentals=N,
            bytes_accessed=4 * (N * N + 3 * N + 1) + 2 * N * w_cols),
    )(words, p2_row, lam)
    return out


# speedup vs baseline: 1.1452x; 1.1452x over previous
import jax
import jax.numpy as jnp
from jax.experimental import pallas as pl
from jax.experimental.pallas import tpu as pltpu


def _p2_kernel(adj_ref, feat_ref, wg_ref, b_ref, w2_ref,
               p2_ref, mask_ref, rowval_ref):
    adj = adj_ref[...]
    t = jnp.dot(adj, feat_ref[...],
                preferred_element_type=jnp.float32)
    h = jnp.dot(t, wg_ref[...],
                preferred_element_type=jnp.float32) + b_ref[...]
    emb = jnp.maximum(h, 0.0)
    p2_ref[...] = jnp.dot(emb, w2_ref[...],
                          preferred_element_type=jnp.float32)
    mask_ref[...] = (adj != 0.0).astype(jnp.int8)
    rowval_ref[...] = jnp.max(adj, axis=1, keepdims=True)


def _combine_kernel(mask_ref, rowval_ref, p2_ref, lam_ref, out_ref):
    maskf = mask_ref[...].astype(jnp.float32)
    p2 = p2_ref[...]
    ep2 = jnp.exp(p2 - jnp.max(p2))
    e = maskf * ep2
    denom = jnp.sum(e, axis=1, keepdims=True)
    scale = lam_ref[0] * pl.reciprocal(jnp.where(denom > 0.0, denom, 1.0))
    out_ref[...] = maskf * rowval_ref[...] + e * scale


def _row_tile(n, cap):
    if n % 8 != 0:
        return n
    tm = min(n, cap)
    tm = max(8, (tm // 8) * 8)
    while tm > 8 and n % tm != 0:
        tm -= 8
    return tm if n % tm == 0 else n


def kernel(v_ori, feat, v_indices, w_gcn, b_gcn, w_mlp, b_mlp, com_lambda):
    del v_indices, b_mlp
    N, F = feat.shape
    H = w_gcn.shape[1]

    tm1 = _row_tile(N, 512)
    tm2 = _row_tile(N, 256)

    cp = pltpu.CompilerParams(dimension_semantics=("parallel",),
                              vmem_limit_bytes=(64 << 20) * 3 // 4)
    vmem_full = pl.BlockSpec(memory_space=pltpu.MemorySpace.VMEM)
    smem_full = pl.BlockSpec(memory_space=pltpu.MemorySpace.SMEM)

    w2 = w_mlp.reshape(2, H)[1].reshape(H, 1)
    b = b_gcn.reshape(1, H)

    p2, mask8, rowval = pl.pallas_call(
        _p2_kernel,
        out_shape=(
            jax.ShapeDtypeStruct((N, 1), jnp.float32),
            jax.ShapeDtypeStruct((N, N), jnp.int8),
            jax.ShapeDtypeStruct((N, 1), jnp.float32),
        ),
        grid=(N // tm1,),
        in_specs=[
            pl.BlockSpec((tm1, N), lambda i: (i, 0)),
            vmem_full,
            vmem_full,
            vmem_full,
            vmem_full,
        ],
        out_specs=(
            pl.BlockSpec((tm1, 1), lambda i: (i, 0)),
            pl.BlockSpec((tm1, N), lambda i: (i, 0)),
            pl.BlockSpec((tm1, 1), lambda i: (i, 0)),
        ),
        compiler_params=cp,
        cost_estimate=pl.CostEstimate(
            flops=2 * N * N * F + 2 * N * F * H + 2 * N * H,
            transcendentals=0,
            bytes_accessed=4 * (N * N + N * F + F * H + N) + N * N),
    )(v_ori, feat, w_gcn, b, w2)

    p2_row = p2.reshape(1, N)
    lam = jnp.asarray(com_lambda, jnp.float32).reshape(1)

    out = pl.pallas_call(
        _combine_kernel,
        out_shape=jax.ShapeDtypeStruct((N, N), jnp.float32),
        grid=(N // tm2,),
        in_specs=[
            pl.BlockSpec((tm2, N), lambda i: (i, 0)),
            pl.BlockSpec((tm2, 1), lambda i: (i, 0)),
            vmem_full,
            smem_full,
        ],
        out_specs=pl.BlockSpec((tm2, N), lambda i: (i, 0)),
        compiler_params=cp,
        cost_estimate=pl.CostEstimate(
            flops=6 * N * N, transcendentals=N,
            bytes_accessed=4 * (N * N + 3 * N + 1) + N * N),
    )(mask8, rowval, p2_row, lam)
    return out


# speedup vs baseline: 1.1974x; 1.0456x over previous
import jax
import jax.numpy as jnp
from jax.experimental import pallas as pl
from jax.experimental.pallas import tpu as pltpu


def _p2_kernel(adj_ref, feat_ref, wg_ref, b_ref, w2_ref,
               p2_ref, mask_ref, rowval_ref):
    adj = adj_ref[...]
    t = jnp.dot(adj, feat_ref[...],
                preferred_element_type=jnp.float32)
    h = jnp.dot(t, wg_ref[...],
                preferred_element_type=jnp.float32) + b_ref[...]
    emb = jnp.maximum(h, 0.0)
    p2_ref[...] = jnp.dot(emb, w2_ref[...],
                          preferred_element_type=jnp.float32)
    mask_ref[...] = (adj != 0.0).astype(jnp.int8)
    rowval_ref[...] = jnp.max(adj, axis=1, keepdims=True)


def _combine_kernel(mask_ref, rowval_ref, p2_ref, lam_ref, out_ref):
    maskf = mask_ref[...].astype(jnp.float32)
    p2 = p2_ref[...]
    ep2 = jnp.exp(p2 - jnp.max(p2))
    e = maskf * ep2
    denom = jnp.sum(e, axis=1, keepdims=True)
    scale = lam_ref[0] * pl.reciprocal(jnp.where(denom > 0.0, denom, 1.0))
    out_ref[...] = maskf * rowval_ref[...] + e * scale


def _row_tile(n, cap):
    if n % 8 != 0:
        return n
    tm = min(n, cap)
    tm = max(8, (tm // 8) * 8)
    while tm > 8 and n % tm != 0:
        tm -= 8
    return tm if n % tm == 0 else n


def kernel(v_ori, feat, v_indices, w_gcn, b_gcn, w_mlp, b_mlp, com_lambda):
    del v_indices, b_mlp
    N, F = feat.shape
    H = w_gcn.shape[1]

    tm1 = _row_tile(N, 512)
    tm2 = _row_tile(N, 512)

    cp = pltpu.CompilerParams(dimension_semantics=("parallel", "arbitrary"),
                              vmem_limit_bytes=(64 << 20) * 3 // 4)
    vmem_full = pl.BlockSpec(memory_space=pltpu.MemorySpace.VMEM)
    smem_full = pl.BlockSpec(memory_space=pltpu.MemorySpace.SMEM)

    w2 = w_mlp.reshape(2, H)[1].reshape(H, 1)
    b = b_gcn.reshape(1, H)

    p2, mask8, rowval = pl.pallas_call(
        _p2_kernel,
        out_shape=(
            jax.ShapeDtypeStruct((N, 1), jnp.float32),
            jax.ShapeDtypeStruct((N, N), jnp.int8),
            jax.ShapeDtypeStruct((N, 1), jnp.float32),
        ),
        grid=(2, N // tm1 // 2),
        in_specs=[
            pl.BlockSpec((tm1, N), lambda c, i: (c * (N // tm1 // 2) + i, 0)),
            vmem_full,
            vmem_full,
            vmem_full,
            vmem_full,
        ],
        out_specs=(
            pl.BlockSpec((tm1, 1), lambda c, i: (c * (N // tm1 // 2) + i, 0)),
            pl.BlockSpec((tm1, N), lambda c, i: (c * (N // tm1 // 2) + i, 0)),
            pl.BlockSpec((tm1, 1), lambda c, i: (c * (N // tm1 // 2) + i, 0)),
        ),
        compiler_params=cp,
        cost_estimate=pl.CostEstimate(
            flops=2 * N * N * F + 2 * N * F * H + 2 * N * H,
            transcendentals=0,
            bytes_accessed=4 * (N * N + N * F + F * H + N) + N * N),
    )(v_ori, feat, w_gcn, b, w2)

    p2_row = p2.reshape(1, N)
    lam = jnp.asarray(com_lambda, jnp.float32).reshape(1)

    out = pl.pallas_call(
        _combine_kernel,
        out_shape=jax.ShapeDtypeStruct((N, N), jnp.float32),
        grid=(2, N // tm2 // 2),
        in_specs=[
            pl.BlockSpec((tm2, N), lambda c, i: (c * (N // tm2 // 2) + i, 0)),
            pl.BlockSpec((tm2, 1), lambda c, i: (c * (N // tm2 // 2) + i, 0)),
            vmem_full,
            smem_full,
        ],
        out_specs=pl.BlockSpec((tm2, N), lambda c, i: (c * (N // tm2 // 2) + i, 0)),
        compiler_params=cp,
        cost_estimate=pl.CostEstimate(
            flops=6 * N * N, transcendentals=N,
            bytes_accessed=4 * (N * N + 3 * N + 1) + N * N),
    )(mask8, rowval, p2_row, lam)
    return out


# speedup vs baseline: 1.2031x; 1.0048x over previous
import jax
import jax.numpy as jnp
from jax.experimental import pallas as pl
from jax.experimental.pallas import tpu as pltpu


def _p2_kernel(adj_ref, feat_ref, wg_ref, b_ref, w2_ref,
               p2_ref, mask_ref, rowval_ref):
    adj = adj_ref[...]
    t = jnp.dot(adj, feat_ref[...],
                preferred_element_type=jnp.float32)
    h = jnp.dot(t, wg_ref[...],
                preferred_element_type=jnp.float32) + b_ref[...]
    emb = jnp.maximum(h, 0.0)
    p2_ref[...] = jnp.dot(emb, w2_ref[...],
                          preferred_element_type=jnp.float32)
    mask_ref[...] = (adj != 0.0).astype(jnp.int8)
    rowval_ref[...] = jnp.max(adj, axis=1, keepdims=True)


def _combine_kernel(mask_ref, rowval_ref, p2_ref, lam_ref, out_ref):
    maskf = mask_ref[...].astype(jnp.float32)
    p2 = p2_ref[...]
    ep2 = jnp.exp(p2 - jnp.max(p2))
    e = maskf * ep2
    denom = jnp.sum(e, axis=1, keepdims=True)
    scale = lam_ref[0] * pl.reciprocal(jnp.where(denom > 0.0, denom, 1.0))
    out_ref[...] = maskf * rowval_ref[...] + e * scale


def _row_tile(n, cap):
    if n % 8 != 0:
        return n
    tm = min(n, cap)
    tm = max(8, (tm // 8) * 8)
    while tm > 8 and n % tm != 0:
        tm -= 8
    return tm if n % tm == 0 else n


def kernel(v_ori, feat, v_indices, w_gcn, b_gcn, w_mlp, b_mlp, com_lambda):
    del v_indices, b_mlp
    N, F = feat.shape
    H = w_gcn.shape[1]

    tm1 = _row_tile(N, 512)
    tm2 = _row_tile(N, 512)

    cp = pltpu.CompilerParams(dimension_semantics=("parallel",),
                              vmem_limit_bytes=(64 << 20) * 3 // 4)
    vmem_full = pl.BlockSpec(memory_space=pltpu.MemorySpace.VMEM)
    smem_full = pl.BlockSpec(memory_space=pltpu.MemorySpace.SMEM)

    w2 = w_mlp.reshape(2, H)[1].reshape(H, 1)
    b = b_gcn.reshape(1, H)

    p2, mask8, rowval = pl.pallas_call(
        _p2_kernel,
        out_shape=(
            jax.ShapeDtypeStruct((N, 1), jnp.float32),
            jax.ShapeDtypeStruct((N, N), jnp.int8),
            jax.ShapeDtypeStruct((N, 1), jnp.float32),
        ),
        grid=(N // tm1,),
        in_specs=[
            pl.BlockSpec((tm1, N), lambda i: (i, 0)),
            vmem_full,
            vmem_full,
            vmem_full,
            vmem_full,
        ],
        out_specs=(
            pl.BlockSpec((tm1, 1), lambda i: (i, 0)),
            pl.BlockSpec((tm1, N), lambda i: (i, 0)),
            pl.BlockSpec((tm1, 1), lambda i: (i, 0)),
        ),
        compiler_params=cp,
        cost_estimate=pl.CostEstimate(
            flops=2 * N * N * F + 2 * N * F * H + 2 * N * H,
            transcendentals=0,
            bytes_accessed=4 * (N * N + N * F + F * H + N) + N * N),
    )(v_ori, feat, w_gcn, b, w2)

    p2_row = p2.reshape(1, N)
    lam = jnp.asarray(com_lambda, jnp.float32).reshape(1)

    out = pl.pallas_call(
        _combine_kernel,
        out_shape=jax.ShapeDtypeStruct((N, N), jnp.float32),
        grid=(N // tm2,),
        in_specs=[
            pl.BlockSpec((tm2, N), lambda i: (i, 0)),
            pl.BlockSpec((tm2, 1), lambda i: (i, 0)),
            vmem_full,
            smem_full,
        ],
        out_specs=pl.BlockSpec((tm2, N), lambda i: (i, 0)),
        compiler_params=cp,
        cost_estimate=pl.CostEstimate(
            flops=6 * N * N, transcendentals=N,
            bytes_accessed=4 * (N * N + 3 * N + 1) + N * N),
    )(mask8, rowval, p2_row, lam)
    return out
